# baseline (speedup 1.0000x reference)
import functools

import jax
import jax.numpy as jnp
from jax import lax
from jax.experimental import pallas as pl
from jax.experimental.pallas import tpu as pltpu
from jax.experimental.pallas import tpu_sc as plsc

NBUF = 4


def _emb_body(n_ch, ch, d, b0, idx_hbm, table_hbm, out_hbm,
              idx_v, gbuf, tbuf, *sems):
    gsem = sems[:NBUF]
    ssem = sems[NBUF:]
    nc = 2
    wid = lax.axis_index("s") * nc + lax.axis_index("c")
    pltpu.sync_copy(idx_hbm.at[wid], idx_v)

    def gather(j, b):
        return pltpu.make_async_copy(table_hbm.at[idx_v.at[j]], gbuf.at[b],
                                     gsem[b])

    def scatters(j, b):
        g0 = (wid * n_ch + j) * ch
        i1 = g0 // b0
        i0b = (g0 % b0) // ch
        base = i1 * (d * (b0 // ch)) + i0b * 8
        return [
            pltpu.make_async_copy(
                tbuf.at[b].at[pl.ds(jt * 8, 8), pl.ds(0, ch)],
                out_hbm.at[pl.ds(base + jt * (8 * (b0 // ch)), 8)],
                ssem[b])
            for jt in range(d // 8)
        ]

    def scatter_start(j, b):
        for c in scatters(j, b):
            c.start()

    def scatter_wait(j, b):
        for c in scatters(j, b):
            c.wait()

    iota = lax.iota(jnp.int32, 16)
    fvs = [iota + (t * 16) for t in range(d // 16)]
    TU = 4

    def transpose(b):
        src = gbuf.at[b]
        dst = tbuf.at[b]

        def tloop(ti, carry):
            tok0 = ti * TU
            for dt in range(TU):
                tok = tok0 + dt
                tokv = jnp.full((16,), tok, jnp.int32)
                row = src.at[tok]
                for t in range(d // 16):
                    v = row[pl.ds(t * 16, 16)]
                    plsc.store_scatter(dst, [fvs[t], tokv], v)
            return carry

        lax.fori_loop(0, ch // TU, tloop, 0)

    for b in range(NBUF):
        gather(b, b).start()

    n_rounds = n_ch // NBUF

    def steady(t, carry):
        j0 = t * NBUF
        for b in range(NBUF):
            j = j0 + b
            gather(j, b).wait()

            @pl.when(t > 0)
            def _():
                scatter_wait(j - NBUF, b)

            transpose(b)
            gather(j + NBUF, b).start()
            scatter_start(j, b)
        return carry

    lax.fori_loop(0, n_rounds - 1, steady, 0)

    j0 = (n_rounds - 1) * NBUF
    for b in range(NBUF):
        j = j0 + b
        gather(j, b).wait()
        scatter_wait(j - NBUF, b)
        transpose(b)
        scatter_start(j, b)
    for b in range(NBUF):
        scatter_wait(j0 + b, b)


def kernel(x, table):
    b0, b1 = x.shape
    v, d = table.shape
    b = b0 * b1
    nw = 32
    ch = 128
    b_per_w = b // nw
    n_ch = b_per_w // ch
    assert b_per_w * nw == b and n_ch * ch == b_per_w and n_ch % NBUF == 0
    assert b0 % ch == 0 and d % 16 == 0

    idx = x.T.reshape(nw, n_ch, ch).astype(jnp.int32)

    n_rows = b * d // ch
    mesh = plsc.VectorSubcoreMesh(core_axis_name="c", subcore_axis_name="s")
    emb = functools.partial(
        pl.kernel,
        mesh=mesh,
        out_type=jax.ShapeDtypeStruct((n_rows, ch), jnp.float32),
        scratch_types=(
            [pltpu.VMEM((n_ch, ch), jnp.int32),
             pltpu.VMEM((NBUF, ch, d), jnp.float32),
             pltpu.VMEM((NBUF, d, ch + 1), jnp.float32)]
            + [pltpu.SemaphoreType.DMA] * (2 * NBUF)
        ),
        compiler_params=pltpu.CompilerParams(use_tc_tiling_on_sc=False,
                                             needs_layout_passes=False),
    )(functools.partial(_emb_body, n_ch, ch, d, b0))

    out = emb(idx, table)
    out6 = out.reshape(b1, d // 8, b0 // ch, 8, ch)
    return out6.transpose(2, 4, 0, 1, 3).reshape(b0, b1, d)
